# baseline (speedup 1.0000x reference)
import jax
import jax.numpy as jnp
from jax import lax
from jax.experimental import pallas as pl
from jax.experimental.pallas import tpu as pltpu
from jax.experimental.pallas import tpu_sc as plsc

BN_EPS = 1e-5

_NC = 2
_NS = 16
_NW = _NC * _NS
_L = 16

_CHUNK = 128


def _sc_segment_sum(embeddings, idx_i32, B, off):
    N, D = embeddings.shape
    N = N - off
    pair_rows = 2 * _CHUNK
    npairs = N // pair_rows
    rem = N - npairs * pair_rows
    rem_full = rem // _CHUNK
    tail = rem - rem_full * _CHUNK
    steps = -(-npairs // _NW)
    rows_per_tile = B // _NS

    mesh = plsc.VectorSubcoreMesh(core_axis_name="c", subcore_axis_name="s")

    def body(emb_hbm, idx_hbm, out_hbm, rows0, rows1, idx0, idx1,
             idx_t, zbuf, acc, sem0, sem1):
        rows_b = (rows0, rows1)
        idx_b = (idx0, idx1)
        sems = (sem0, sem1)
        c = lax.axis_index("c")
        s = lax.axis_index("s")
        wid = c * _NS + s

        for r in range(rows_per_tile):
            for j in range(D // _L):
                zbuf[r, pl.ds(j * _L, _L)] = jnp.zeros((_L,), jnp.float32)
        pltpu.sync_copy(zbuf, acc.at[pl.ds(s * rows_per_tile, rows_per_tile)])
        plsc.subcore_barrier()

        def gather_descs(pid, b):
            base = pl.multiple_of(off + pid * pair_rows, 8)
            return (
                (emb_hbm.at[pl.ds(base, pair_rows)], rows_b[b]),
                (idx_hbm.at[pl.ds(base, _CHUNK)], idx_b[b].at[0]),
                (idx_hbm.at[pl.ds(base + _CHUNK, _CHUNK)], idx_b[b].at[1]),
            )

        def issue(t, b):
            pid = wid + _NW * t

            @pl.when(pid < npairs)
            def _():
                for src, dst in gather_descs(pid, b):
                    pltpu.async_copy(src, dst, sems[b])

        issue(0, 0)
        for t in range(steps):
            b = t % 2
            if t + 1 < steps:
                issue(t + 1, 1 - b)
            pid = wid + _NW * t

            @pl.when(pid < npairs)
            def _():
                for src, dst in gather_descs(pid, b):
                    pltpu.make_async_copy(src, dst, sems[b]).wait()
                pltpu.sync_copy(rows_b[b].at[pl.ds(0, _CHUNK)],
                                acc.at[idx_b[b].at[0]], add=True)
                pltpu.sync_copy(rows_b[b].at[pl.ds(_CHUNK, _CHUNK)],
                                acc.at[idx_b[b].at[1]], add=True)

        if rem_full:
            @pl.when(wid == _NW - 2)
            def _():
                base = off + npairs * pair_rows
                pltpu.sync_copy(idx_hbm.at[pl.ds(base, _CHUNK)], idx0.at[0])
                pltpu.sync_copy(emb_hbm.at[pl.ds(base, _CHUNK)],
                                rows0.at[pl.ds(0, _CHUNK)])
                pltpu.sync_copy(rows0.at[pl.ds(0, _CHUNK)],
                                acc.at[idx0.at[0]], add=True)

        if tail:
            @pl.when(wid == _NW - 1)
            def _():
                base = off + npairs * pair_rows + rem_full * _CHUNK
                pltpu.sync_copy(idx_hbm.at[pl.ds(base, tail)], idx_t)
                pltpu.sync_copy(emb_hbm.at[pl.ds(base, tail)],
                                rows1.at[pl.ds(0, tail)])
                pltpu.sync_copy(rows1.at[pl.ds(0, tail)],
                                acc.at[idx_t], add=True)

        plsc.subcore_barrier()
        row0 = s * rows_per_tile
        pltpu.sync_copy(acc.at[pl.ds(row0, rows_per_tile)],
                        out_hbm.at[c, pl.ds(row0, rows_per_tile)])

    k = pl.kernel(
        body,
        out_type=jax.ShapeDtypeStruct((_NC, B, D), jnp.float32),
        mesh=mesh,
        scratch_types=[
            pltpu.VMEM((pair_rows, D), jnp.float32),
            pltpu.VMEM((pair_rows, D), jnp.float32),
            pltpu.VMEM((2, _CHUNK), jnp.int32),
            pltpu.VMEM((2, _CHUNK), jnp.int32),
            pltpu.VMEM((max(tail, 8),), jnp.int32),
            pltpu.VMEM((rows_per_tile, D), jnp.float32),
            pltpu.VMEM_SHARED((B, D), jnp.float32),
            pltpu.SemaphoreType.DMA,
            pltpu.SemaphoreType.DMA,
        ],
    )
    return k(embeddings, idx_i32)


_TC_BLK = 1024


def _tc_partial_segsum(embeddings, idx2d, B, nrows):
    N, D = embeddings.shape
    nblk = nrows // _TC_BLK

    def body(idx_ref, emb_ref, o_ref):
        @pl.when(pl.program_id(0) == 0)
        def _():
            o_ref[...] = jnp.zeros_like(o_ref)

        seg = lax.broadcasted_iota(jnp.int32, (B, _TC_BLK), 0)
        onehot = (seg == idx_ref[...]).astype(jnp.bfloat16)
        rows = emb_ref[...].astype(jnp.bfloat16)
        o_ref[...] += lax.dot_general(
            onehot, rows, (((1,), (0,)), ((), ())),
            preferred_element_type=jnp.float32)

    return pl.pallas_call(
        body,
        grid=(nblk,),
        in_specs=[
            pl.BlockSpec((1, _TC_BLK), lambda i: (0, i)),
            pl.BlockSpec((_TC_BLK, D), lambda i: (i, 0)),
        ],
        out_specs=pl.BlockSpec((B, D), lambda i: (0, 0)),
        out_shape=jax.ShapeDtypeStruct((B, D), jnp.float32),
    )(idx2d, embeddings)


def _tc_mlp(partials, tc_partial, virtual_node, W1, b1, g1, be1,
            W2, b2, g2, be2):
    B, D = virtual_node.shape

    def body(p_ref, pt_ref, vn_ref, w1_ref, b1_ref, g1_ref, be1_ref,
             w2_ref, b2_ref, g2_ref, be2_ref, o_ref):
        vn = vn_ref[...] + (p_ref[0] + p_ref[1] + pt_ref[...])
        h = lax.dot_general(vn, w1_ref[...], (((1,), (0,)), ((), ())),
                            precision=lax.Precision.HIGHEST,
                            preferred_element_type=jnp.float32)
        h = h + b1_ref[...]
        mu = jnp.mean(h, axis=0, keepdims=True)
        var = jnp.mean((h - mu) * (h - mu), axis=0, keepdims=True)
        h = (h - mu) / jnp.sqrt(var + BN_EPS) * g1_ref[...] + be1_ref[...]
        h = jnp.maximum(h, 0.0)
        h = lax.dot_general(h, w2_ref[...], (((1,), (0,)), ((), ())),
                            precision=lax.Precision.HIGHEST,
                            preferred_element_type=jnp.float32)
        h = h + b2_ref[...]
        mu2 = jnp.mean(h, axis=0, keepdims=True)
        var2 = jnp.mean((h - mu2) * (h - mu2), axis=0, keepdims=True)
        h = (h - mu2) / jnp.sqrt(var2 + BN_EPS) * g2_ref[...] + be2_ref[...]
        o_ref[...] = jnp.maximum(h, 0.0)

    return pl.pallas_call(
        body,
        out_shape=jax.ShapeDtypeStruct((B, D), jnp.float32),
    )(partials, tc_partial, virtual_node, W1, b1.reshape(1, -1),
      g1.reshape(1, -1), be1.reshape(1, -1), W2, b2.reshape(1, -1),
      g2.reshape(1, -1), be2.reshape(1, -1))


_TC_NBLK = 65


def kernel(virtual_node, embeddings, batch_idx, W1, b1, g1, be1,
           W2, b2, g2, be2):
    B = virtual_node.shape[0]
    N = embeddings.shape[0]
    idx_i32 = batch_idx.astype(jnp.int32)
    tc_rows = min(_TC_NBLK * _TC_BLK, N // _TC_BLK * _TC_BLK)
    sc_partials = _sc_segment_sum(embeddings, idx_i32, B, tc_rows)
    tc_partial = _tc_partial_segsum(embeddings, idx_i32.reshape(1, -1), B,
                                    tc_rows)
    return _tc_mlp(sc_partials, tc_partial, virtual_node, W1, b1, g1, be1,
                   W2, b2, g2, be2)


# speedup vs baseline: 1.4428x; 1.4428x over previous
import jax
import jax.numpy as jnp
from jax import lax
from jax.experimental import pallas as pl
from jax.experimental.pallas import tpu as pltpu
from jax.experimental.pallas import tpu_sc as plsc

BN_EPS = 1e-5

_NC = 2
_NS = 16
_NW = _NC * _NS
_L = 16

_CHUNK = 128


def _sc_segment_sum(embeddings, idx_i32, B, off):
    N, D = embeddings.shape
    N = N - off
    pair_rows = 2 * _CHUNK
    npairs = N // pair_rows
    rem = N - npairs * pair_rows
    rem_full = rem // _CHUNK
    tail = rem - rem_full * _CHUNK
    steps = -(-npairs // _NW)
    rows_per_tile = B // _NS

    mesh = plsc.VectorSubcoreMesh(core_axis_name="c", subcore_axis_name="s")

    def body(emb_hbm, idx_hbm, out_hbm, rows0, rows1, idx0, idx1,
             idx_t, zbuf, acc, sem0, sem1):
        rows_b = (rows0, rows1)
        idx_b = (idx0, idx1)
        sems = (sem0, sem1)
        c = lax.axis_index("c")
        s = lax.axis_index("s")
        wid = c * _NS + s

        for r in range(rows_per_tile):
            for j in range(D // _L):
                zbuf[r, pl.ds(j * _L, _L)] = jnp.zeros((_L,), jnp.float32)
        pltpu.sync_copy(zbuf, acc.at[pl.ds(s * rows_per_tile, rows_per_tile)])
        plsc.subcore_barrier()

        def gather_descs(pid, b):
            base = pl.multiple_of(off + pid * pair_rows, 8)
            return (
                (emb_hbm.at[pl.ds(base, pair_rows)], rows_b[b]),
                (idx_hbm.at[pl.ds(base, _CHUNK)], idx_b[b].at[0]),
                (idx_hbm.at[pl.ds(base + _CHUNK, _CHUNK)], idx_b[b].at[1]),
            )

        def issue(t, b):
            pid = wid + _NW * t

            @pl.when(pid < npairs)
            def _():
                for src, dst in gather_descs(pid, b):
                    pltpu.async_copy(src, dst, sems[b])

        issue(0, 0)
        for t in range(steps):
            b = t % 2
            if t + 1 < steps:
                issue(t + 1, 1 - b)
            pid = wid + _NW * t

            @pl.when(pid < npairs)
            def _():
                for src, dst in gather_descs(pid, b):
                    pltpu.make_async_copy(src, dst, sems[b]).wait()
                pltpu.sync_copy(rows_b[b].at[pl.ds(0, _CHUNK)],
                                acc.at[idx_b[b].at[0]], add=True)
                pltpu.sync_copy(rows_b[b].at[pl.ds(_CHUNK, _CHUNK)],
                                acc.at[idx_b[b].at[1]], add=True)

        if rem_full:
            @pl.when(wid == _NW - 2)
            def _():
                base = off + npairs * pair_rows
                pltpu.sync_copy(idx_hbm.at[pl.ds(base, _CHUNK)], idx0.at[0])
                pltpu.sync_copy(emb_hbm.at[pl.ds(base, _CHUNK)],
                                rows0.at[pl.ds(0, _CHUNK)])
                pltpu.sync_copy(rows0.at[pl.ds(0, _CHUNK)],
                                acc.at[idx0.at[0]], add=True)

        if tail:
            @pl.when(wid == _NW - 1)
            def _():
                base = off + npairs * pair_rows + rem_full * _CHUNK
                pltpu.sync_copy(idx_hbm.at[pl.ds(base, tail)], idx_t)
                pltpu.sync_copy(emb_hbm.at[pl.ds(base, tail)],
                                rows1.at[pl.ds(0, tail)])
                pltpu.sync_copy(rows1.at[pl.ds(0, tail)],
                                acc.at[idx_t], add=True)

        plsc.subcore_barrier()
        row0 = s * rows_per_tile
        pltpu.sync_copy(acc.at[pl.ds(row0, rows_per_tile)],
                        out_hbm.at[c, pl.ds(row0, rows_per_tile)])

    k = pl.kernel(
        body,
        out_type=jax.ShapeDtypeStruct((_NC, B, D), jnp.float32),
        mesh=mesh,
        scratch_types=[
            pltpu.VMEM((pair_rows, D), jnp.float32),
            pltpu.VMEM((pair_rows, D), jnp.float32),
            pltpu.VMEM((2, _CHUNK), jnp.int32),
            pltpu.VMEM((2, _CHUNK), jnp.int32),
            pltpu.VMEM((max(tail, 8),), jnp.int32),
            pltpu.VMEM((rows_per_tile, D), jnp.float32),
            pltpu.VMEM_SHARED((B, D), jnp.float32),
            pltpu.SemaphoreType.DMA,
            pltpu.SemaphoreType.DMA,
        ],
    )
    return k(embeddings, idx_i32)


_TC_BLK = 1024


def _tc_partial_segsum(embeddings, idx2d, B, nrows):
    N, D = embeddings.shape
    nblk = nrows // _TC_BLK

    def body(idx_ref, emb_ref, o_ref):
        @pl.when(pl.program_id(0) == 0)
        def _():
            o_ref[...] = jnp.zeros_like(o_ref)

        seg = lax.broadcasted_iota(jnp.int32, (B, _TC_BLK), 0)
        onehot = (seg == idx_ref[...]).astype(jnp.bfloat16)
        rows = emb_ref[...].astype(jnp.bfloat16)
        o_ref[...] += lax.dot_general(
            onehot, rows, (((1,), (0,)), ((), ())),
            preferred_element_type=jnp.float32)

    return pl.pallas_call(
        body,
        grid=(nblk,),
        in_specs=[
            pl.BlockSpec((1, _TC_BLK), lambda i: (0, i)),
            pl.BlockSpec((_TC_BLK, D), lambda i: (i, 0)),
        ],
        out_specs=pl.BlockSpec((B, D), lambda i: (0, 0)),
        out_shape=jax.ShapeDtypeStruct((B, D), jnp.float32),
    )(idx2d, embeddings)


def _tc_mlp(partials, tc_partial, virtual_node, W1, b1, g1, be1,
            W2, b2, g2, be2):
    B, D = virtual_node.shape

    def body(p_ref, pt_ref, vn_ref, w1_ref, b1_ref, g1_ref, be1_ref,
             w2_ref, b2_ref, g2_ref, be2_ref, o_ref):
        vn = vn_ref[...] + (p_ref[0] + p_ref[1] + pt_ref[...])
        h = lax.dot_general(vn, w1_ref[...], (((1,), (0,)), ((), ())),
                            precision=lax.Precision.HIGHEST,
                            preferred_element_type=jnp.float32)
        h = h + b1_ref[...]
        mu = jnp.mean(h, axis=0, keepdims=True)
        var = jnp.mean((h - mu) * (h - mu), axis=0, keepdims=True)
        h = (h - mu) / jnp.sqrt(var + BN_EPS) * g1_ref[...] + be1_ref[...]
        h = jnp.maximum(h, 0.0)
        h = lax.dot_general(h, w2_ref[...], (((1,), (0,)), ((), ())),
                            precision=lax.Precision.HIGHEST,
                            preferred_element_type=jnp.float32)
        h = h + b2_ref[...]
        mu2 = jnp.mean(h, axis=0, keepdims=True)
        var2 = jnp.mean((h - mu2) * (h - mu2), axis=0, keepdims=True)
        h = (h - mu2) / jnp.sqrt(var2 + BN_EPS) * g2_ref[...] + be2_ref[...]
        o_ref[...] = jnp.maximum(h, 0.0)

    return pl.pallas_call(
        body,
        out_shape=jax.ShapeDtypeStruct((B, D), jnp.float32),
    )(partials, tc_partial, virtual_node, W1, b1.reshape(1, -1),
      g1.reshape(1, -1), be1.reshape(1, -1), W2, b2.reshape(1, -1),
      g2.reshape(1, -1), be2.reshape(1, -1))


_TC_NBLK = 33


def kernel(virtual_node, embeddings, batch_idx, W1, b1, g1, be1,
           W2, b2, g2, be2):
    B = virtual_node.shape[0]
    N = embeddings.shape[0]
    idx_i32 = batch_idx.astype(jnp.int32)
    tc_rows = min(_TC_NBLK * _TC_BLK, N // _TC_BLK * _TC_BLK)
    sc_partials = _sc_segment_sum(embeddings, idx_i32, B, tc_rows)
    tc_partial = _tc_partial_segsum(embeddings, idx_i32.reshape(1, -1), B,
                                    tc_rows)
    return _tc_mlp(sc_partials, tc_partial, virtual_node, W1, b1, g1, be1,
                   W2, b2, g2, be2)


# speedup vs baseline: 1.5331x; 1.0626x over previous
import jax
import jax.numpy as jnp
from jax import lax
from jax.experimental import pallas as pl
from jax.experimental.pallas import tpu as pltpu
from jax.experimental.pallas import tpu_sc as plsc

BN_EPS = 1e-5

_NC = 2
_NS = 16
_NW = _NC * _NS
_L = 16

_CHUNK = 128


def _sc_segment_sum(embeddings, idx_i32, B, off):
    N, D = embeddings.shape
    N = N - off
    pair_rows = 2 * _CHUNK
    npairs = N // pair_rows
    rem = N - npairs * pair_rows
    rem_full = rem // _CHUNK
    tail = rem - rem_full * _CHUNK
    steps = -(-npairs // _NW)
    rows_per_tile = B // _NS

    mesh = plsc.VectorSubcoreMesh(core_axis_name="c", subcore_axis_name="s")

    def body(emb_hbm, idx_hbm, out_hbm, rows0, rows1, idx0, idx1,
             idx_t, zbuf, acc, sem0, sem1):
        rows_b = (rows0, rows1)
        idx_b = (idx0, idx1)
        sems = (sem0, sem1)
        c = lax.axis_index("c")
        s = lax.axis_index("s")
        wid = c * _NS + s

        for r in range(rows_per_tile):
            for j in range(D // _L):
                zbuf[r, pl.ds(j * _L, _L)] = jnp.zeros((_L,), jnp.float32)
        pltpu.sync_copy(zbuf, acc.at[pl.ds(s * rows_per_tile, rows_per_tile)])
        plsc.subcore_barrier()

        def gather_descs(pid, b):
            base = pl.multiple_of(off + pid * pair_rows, 8)
            return (
                (emb_hbm.at[pl.ds(base, pair_rows)], rows_b[b]),
                (idx_hbm.at[pl.ds(base, _CHUNK)], idx_b[b].at[0]),
                (idx_hbm.at[pl.ds(base + _CHUNK, _CHUNK)], idx_b[b].at[1]),
            )

        def issue(t, b):
            pid = wid + _NW * t

            @pl.when(pid < npairs)
            def _():
                for src, dst in gather_descs(pid, b):
                    pltpu.async_copy(src, dst, sems[b])

        issue(0, 0)
        for t in range(steps):
            b = t % 2
            if t + 1 < steps:
                issue(t + 1, 1 - b)
            pid = wid + _NW * t

            @pl.when(pid < npairs)
            def _():
                for src, dst in gather_descs(pid, b):
                    pltpu.make_async_copy(src, dst, sems[b]).wait()
                pltpu.sync_copy(rows_b[b].at[pl.ds(0, _CHUNK)],
                                acc.at[idx_b[b].at[0]], add=True)
                pltpu.sync_copy(rows_b[b].at[pl.ds(_CHUNK, _CHUNK)],
                                acc.at[idx_b[b].at[1]], add=True)

        if rem_full:
            @pl.when(wid == _NW - 2)
            def _():
                base = off + npairs * pair_rows
                pltpu.sync_copy(idx_hbm.at[pl.ds(base, _CHUNK)], idx0.at[0])
                pltpu.sync_copy(emb_hbm.at[pl.ds(base, _CHUNK)],
                                rows0.at[pl.ds(0, _CHUNK)])
                pltpu.sync_copy(rows0.at[pl.ds(0, _CHUNK)],
                                acc.at[idx0.at[0]], add=True)

        if tail:
            @pl.when(wid == _NW - 1)
            def _():
                base = off + npairs * pair_rows + rem_full * _CHUNK
                pltpu.sync_copy(idx_hbm.at[pl.ds(base, tail)], idx_t)
                pltpu.sync_copy(emb_hbm.at[pl.ds(base, tail)],
                                rows1.at[pl.ds(0, tail)])
                pltpu.sync_copy(rows1.at[pl.ds(0, tail)],
                                acc.at[idx_t], add=True)

        plsc.subcore_barrier()
        row0 = s * rows_per_tile
        pltpu.sync_copy(acc.at[pl.ds(row0, rows_per_tile)],
                        out_hbm.at[c, pl.ds(row0, rows_per_tile)])

    k = pl.kernel(
        body,
        out_type=jax.ShapeDtypeStruct((_NC, B, D), jnp.float32),
        mesh=mesh,
        scratch_types=[
            pltpu.VMEM((pair_rows, D), jnp.float32),
            pltpu.VMEM((pair_rows, D), jnp.float32),
            pltpu.VMEM((2, _CHUNK), jnp.int32),
            pltpu.VMEM((2, _CHUNK), jnp.int32),
            pltpu.VMEM((max(tail, 8),), jnp.int32),
            pltpu.VMEM((rows_per_tile, D), jnp.float32),
            pltpu.VMEM_SHARED((B, D), jnp.float32),
            pltpu.SemaphoreType.DMA,
            pltpu.SemaphoreType.DMA,
        ],
    )
    return k(embeddings, idx_i32)


_TC_BLK = 2048


def _tc_partial_segsum(embeddings, idx2d, B, nrows):
    N, D = embeddings.shape
    nblk = nrows // _TC_BLK

    def body(idx_ref, emb_ref, o_ref, acc_ref):
        i = pl.program_id(0)

        @pl.when(i == 0)
        def _():
            acc_ref[...] = jnp.zeros_like(acc_ref)

        seg = lax.broadcasted_iota(jnp.int32, (B, _TC_BLK), 0)
        onehot = (seg == idx_ref[...]).astype(jnp.float32)
        acc_ref[...] += lax.dot_general(
            onehot, emb_ref[...], (((1,), (0,)), ((), ())),
            precision=lax.Precision.DEFAULT,
            preferred_element_type=jnp.float32)

        @pl.when(i == nblk - 1)
        def _():
            o_ref[...] = acc_ref[...]

    return pl.pallas_call(
        body,
        grid=(nblk,),
        in_specs=[
            pl.BlockSpec((1, _TC_BLK), lambda i: (0, i)),
            pl.BlockSpec((_TC_BLK, D), lambda i: (i, 0)),
        ],
        out_specs=pl.BlockSpec((B, D), lambda i: (0, 0)),
        out_shape=jax.ShapeDtypeStruct((B, D), jnp.float32),
        scratch_shapes=[pltpu.VMEM((B, D), jnp.float32)],
    )(idx2d, embeddings)


def _tc_mlp(partials, tc_partial, virtual_node, W1, b1, g1, be1,
            W2, b2, g2, be2):
    B, D = virtual_node.shape

    def body(p_ref, pt_ref, vn_ref, w1_ref, b1_ref, g1_ref, be1_ref,
             w2_ref, b2_ref, g2_ref, be2_ref, o_ref):
        vn = vn_ref[...] + (p_ref[0] + p_ref[1] + pt_ref[...])
        h = lax.dot_general(vn, w1_ref[...], (((1,), (0,)), ((), ())),
                            precision=lax.Precision.HIGHEST,
                            preferred_element_type=jnp.float32)
        h = h + b1_ref[...]
        mu = jnp.mean(h, axis=0, keepdims=True)
        var = jnp.mean((h - mu) * (h - mu), axis=0, keepdims=True)
        h = (h - mu) / jnp.sqrt(var + BN_EPS) * g1_ref[...] + be1_ref[...]
        h = jnp.maximum(h, 0.0)
        h = lax.dot_general(h, w2_ref[...], (((1,), (0,)), ((), ())),
                            precision=lax.Precision.HIGHEST,
                            preferred_element_type=jnp.float32)
        h = h + b2_ref[...]
        mu2 = jnp.mean(h, axis=0, keepdims=True)
        var2 = jnp.mean((h - mu2) * (h - mu2), axis=0, keepdims=True)
        h = (h - mu2) / jnp.sqrt(var2 + BN_EPS) * g2_ref[...] + be2_ref[...]
        o_ref[...] = jnp.maximum(h, 0.0)

    return pl.pallas_call(
        body,
        out_shape=jax.ShapeDtypeStruct((B, D), jnp.float32),
    )(partials, tc_partial, virtual_node, W1, b1.reshape(1, -1),
      g1.reshape(1, -1), be1.reshape(1, -1), W2, b2.reshape(1, -1),
      g2.reshape(1, -1), be2.reshape(1, -1))


_TC_NBLK = 17


def kernel(virtual_node, embeddings, batch_idx, W1, b1, g1, be1,
           W2, b2, g2, be2):
    B = virtual_node.shape[0]
    N = embeddings.shape[0]
    idx_i32 = batch_idx.astype(jnp.int32)
    tc_rows = min(_TC_NBLK * _TC_BLK, N // _TC_BLK * _TC_BLK)
    sc_partials = _sc_segment_sum(embeddings, idx_i32, B, tc_rows)
    tc_partial = _tc_partial_segsum(embeddings, idx_i32.reshape(1, -1), B,
                                    tc_rows)
    return _tc_mlp(sc_partials, tc_partial, virtual_node, W1, b1, g1, be1,
                   W2, b2, g2, be2)


# speedup vs baseline: 1.5660x; 1.0215x over previous
import jax
import jax.numpy as jnp
from jax import lax
from jax.experimental import pallas as pl
from jax.experimental.pallas import tpu as pltpu
from jax.experimental.pallas import tpu_sc as plsc

BN_EPS = 1e-5

_NC = 2
_NS = 16
_NW = _NC * _NS
_L = 16

_CHUNK = 128


def _sc_segment_sum(embeddings, idx_i32, B, off):
    N, D = embeddings.shape
    N = N - off
    pair_rows = 2 * _CHUNK
    npairs = N // pair_rows
    rem = N - npairs * pair_rows
    rem_full = rem // _CHUNK
    tail = rem - rem_full * _CHUNK
    steps = -(-npairs // _NW)
    rows_per_tile = B // _NS

    mesh = plsc.VectorSubcoreMesh(core_axis_name="c", subcore_axis_name="s")

    def body(emb_hbm, idx_hbm, out_hbm, rows0, rows1, idx0, idx1,
             idx_t, zbuf, acc, sem0, sem1):
        rows_b = (rows0, rows1)
        idx_b = (idx0, idx1)
        sems = (sem0, sem1)
        c = lax.axis_index("c")
        s = lax.axis_index("s")
        wid = c * _NS + s

        for r in range(rows_per_tile):
            for j in range(D // _L):
                zbuf[r, pl.ds(j * _L, _L)] = jnp.zeros((_L,), jnp.float32)
        pltpu.sync_copy(zbuf, acc.at[pl.ds(s * rows_per_tile, rows_per_tile)])
        plsc.subcore_barrier()

        def gather_descs(pid, b):
            base = pl.multiple_of(off + pid * pair_rows, 8)
            return (
                (emb_hbm.at[pl.ds(base, pair_rows)], rows_b[b]),
                (idx_hbm.at[pl.ds(base, _CHUNK)], idx_b[b].at[0]),
                (idx_hbm.at[pl.ds(base + _CHUNK, _CHUNK)], idx_b[b].at[1]),
            )

        def issue(t, b):
            pid = wid + _NW * t

            @pl.when(pid < npairs)
            def _():
                for src, dst in gather_descs(pid, b):
                    pltpu.async_copy(src, dst, sems[b])

        issue(0, 0)
        for t in range(steps):
            b = t % 2
            if t + 1 < steps:
                issue(t + 1, 1 - b)
            pid = wid + _NW * t

            @pl.when(pid < npairs)
            def _():
                for src, dst in gather_descs(pid, b):
                    pltpu.make_async_copy(src, dst, sems[b]).wait()
                pltpu.sync_copy(rows_b[b].at[pl.ds(0, _CHUNK)],
                                acc.at[idx_b[b].at[0]], add=True)
                pltpu.sync_copy(rows_b[b].at[pl.ds(_CHUNK, _CHUNK)],
                                acc.at[idx_b[b].at[1]], add=True)

        if rem_full:
            @pl.when(wid == _NW - 2)
            def _():
                base = off + npairs * pair_rows
                pltpu.sync_copy(idx_hbm.at[pl.ds(base, _CHUNK)], idx0.at[0])
                pltpu.sync_copy(emb_hbm.at[pl.ds(base, _CHUNK)],
                                rows0.at[pl.ds(0, _CHUNK)])
                pltpu.sync_copy(rows0.at[pl.ds(0, _CHUNK)],
                                acc.at[idx0.at[0]], add=True)

        if tail:
            @pl.when(wid == _NW - 1)
            def _():
                base = off + npairs * pair_rows + rem_full * _CHUNK
                pltpu.sync_copy(idx_hbm.at[pl.ds(base, tail)], idx_t)
                pltpu.sync_copy(emb_hbm.at[pl.ds(base, tail)],
                                rows1.at[pl.ds(0, tail)])
                pltpu.sync_copy(rows1.at[pl.ds(0, tail)],
                                acc.at[idx_t], add=True)

        plsc.subcore_barrier()
        row0 = s * rows_per_tile
        pltpu.sync_copy(acc.at[pl.ds(row0, rows_per_tile)],
                        out_hbm.at[c, pl.ds(row0, rows_per_tile)])

    k = pl.kernel(
        body,
        out_type=jax.ShapeDtypeStruct((_NC, B, D), jnp.float32),
        mesh=mesh,
        scratch_types=[
            pltpu.VMEM((pair_rows, D), jnp.float32),
            pltpu.VMEM((pair_rows, D), jnp.float32),
            pltpu.VMEM((2, _CHUNK), jnp.int32),
            pltpu.VMEM((2, _CHUNK), jnp.int32),
            pltpu.VMEM((max(tail, 8),), jnp.int32),
            pltpu.VMEM((rows_per_tile, D), jnp.float32),
            pltpu.VMEM_SHARED((B, D), jnp.float32),
            pltpu.SemaphoreType.DMA,
            pltpu.SemaphoreType.DMA,
        ],
    )
    return k(embeddings, idx_i32)


_TC_BLK = 2048


def _tc_partial_segsum(embeddings, idx2d, B, nrows):
    N, D = embeddings.shape
    nblk = nrows // _TC_BLK

    def body(idx_ref, emb_ref, o_ref, acc_ref):
        i = pl.program_id(0)

        @pl.when(i == 0)
        def _():
            acc_ref[...] = jnp.zeros_like(acc_ref)

        seg = lax.broadcasted_iota(jnp.int32, (B, _TC_BLK), 0)
        onehot = (seg == idx_ref[...]).astype(jnp.float32)
        acc_ref[...] += lax.dot_general(
            onehot, emb_ref[...], (((1,), (0,)), ((), ())),
            precision=lax.Precision.DEFAULT,
            preferred_element_type=jnp.float32)

        @pl.when(i == nblk - 1)
        def _():
            o_ref[...] = acc_ref[...]

    return pl.pallas_call(
        body,
        grid=(nblk,),
        in_specs=[
            pl.BlockSpec((1, _TC_BLK), lambda i: (0, i)),
            pl.BlockSpec((_TC_BLK, D), lambda i: (i, 0)),
        ],
        out_specs=pl.BlockSpec((B, D), lambda i: (0, 0)),
        out_shape=jax.ShapeDtypeStruct((B, D), jnp.float32),
        scratch_shapes=[pltpu.VMEM((B, D), jnp.float32)],
    )(idx2d, embeddings)


def _tc_mlp(partials, tc_partial, virtual_node, W1, b1, g1, be1,
            W2, b2, g2, be2):
    B, D = virtual_node.shape

    def body(p_ref, pt_ref, vn_ref, w1_ref, b1_ref, g1_ref, be1_ref,
             w2_ref, b2_ref, g2_ref, be2_ref, o_ref):
        vn = vn_ref[...] + (p_ref[0] + p_ref[1] + pt_ref[...])
        h = lax.dot_general(vn, w1_ref[...], (((1,), (0,)), ((), ())),
                            precision=lax.Precision.DEFAULT,
                            preferred_element_type=jnp.float32)
        h = h + b1_ref[...]
        mu = jnp.mean(h, axis=0, keepdims=True)
        var = jnp.mean((h - mu) * (h - mu), axis=0, keepdims=True)
        h = (h - mu) / jnp.sqrt(var + BN_EPS) * g1_ref[...] + be1_ref[...]
        h = jnp.maximum(h, 0.0)
        h = lax.dot_general(h, w2_ref[...], (((1,), (0,)), ((), ())),
                            precision=lax.Precision.DEFAULT,
                            preferred_element_type=jnp.float32)
        h = h + b2_ref[...]
        mu2 = jnp.mean(h, axis=0, keepdims=True)
        var2 = jnp.mean((h - mu2) * (h - mu2), axis=0, keepdims=True)
        h = (h - mu2) / jnp.sqrt(var2 + BN_EPS) * g2_ref[...] + be2_ref[...]
        o_ref[...] = jnp.maximum(h, 0.0)

    return pl.pallas_call(
        body,
        out_shape=jax.ShapeDtypeStruct((B, D), jnp.float32),
    )(partials, tc_partial, virtual_node, W1, b1.reshape(1, -1),
      g1.reshape(1, -1), be1.reshape(1, -1), W2, b2.reshape(1, -1),
      g2.reshape(1, -1), be2.reshape(1, -1))


_TC_NBLK = 20


def kernel(virtual_node, embeddings, batch_idx, W1, b1, g1, be1,
           W2, b2, g2, be2):
    B = virtual_node.shape[0]
    N = embeddings.shape[0]
    idx_i32 = batch_idx.astype(jnp.int32)
    tc_rows = min(_TC_NBLK * _TC_BLK, N // _TC_BLK * _TC_BLK)
    sc_partials = _sc_segment_sum(embeddings, idx_i32, B, tc_rows)
    tc_partial = _tc_partial_segsum(embeddings, idx_i32.reshape(1, -1), B,
                                    tc_rows)
    return _tc_mlp(sc_partials, tc_partial, virtual_node, W1, b1, g1, be1,
                   W2, b2, g2, be2)


# speedup vs baseline: 1.5854x; 1.0124x over previous
import jax
import jax.numpy as jnp
from jax import lax
from jax.experimental import pallas as pl
from jax.experimental.pallas import tpu as pltpu
from jax.experimental.pallas import tpu_sc as plsc

BN_EPS = 1e-5

_NC = 2
_NS = 16
_NW = _NC * _NS
_L = 16

_CHUNK = 128


def _sc_segment_sum(embeddings, idx_i32, B, off):
    N, D = embeddings.shape
    N = N - off
    pair_rows = 2 * _CHUNK
    npairs = N // pair_rows
    rem = N - npairs * pair_rows
    rem_full = rem // _CHUNK
    tail = rem - rem_full * _CHUNK
    steps = -(-npairs // _NW)
    rows_per_tile = B // _NS

    mesh = plsc.VectorSubcoreMesh(core_axis_name="c", subcore_axis_name="s")

    def body(emb_hbm, idx_hbm, out_hbm, rows0, rows1, idx0, idx1,
             idx_t, zbuf, acc, sem0, sem1):
        rows_b = (rows0, rows1)
        idx_b = (idx0, idx1)
        sems = (sem0, sem1)
        c = lax.axis_index("c")
        s = lax.axis_index("s")
        wid = c * _NS + s

        @pl.loop(0, rows_per_tile)
        def _(r):
            for j in range(D // _L):
                zbuf[r, pl.ds(j * _L, _L)] = jnp.zeros((_L,), jnp.float32)

        pltpu.sync_copy(zbuf, acc.at[pl.ds(s * rows_per_tile, rows_per_tile)])
        plsc.subcore_barrier()

        def gather_descs(pid, b):
            base = pl.multiple_of(off + pid * pair_rows, 8)
            return (
                (emb_hbm.at[pl.ds(base, pair_rows)], rows_b[b]),
                (idx_hbm.at[pl.ds(base, _CHUNK)], idx_b[b].at[0]),
                (idx_hbm.at[pl.ds(base + _CHUNK, _CHUNK)], idx_b[b].at[1]),
            )

        def issue(t, b):
            pid = wid + _NW * t

            @pl.when(pid < npairs)
            def _():
                for src, dst in gather_descs(pid, b):
                    pltpu.async_copy(src, dst, sems[b])

        issue(0, 0)
        issue(1, 1)

        @pl.loop(0, 2 * (-(-steps // 2)), step=2)
        def _(t):
            for dt in range(2):
                b = dt
                pid = wid + _NW * (t + dt)

                @pl.when(pid < npairs)
                def _():
                    for src, dst in gather_descs(pid, b):
                        pltpu.make_async_copy(src, dst, sems[b]).wait()
                    pltpu.sync_copy(rows_b[b].at[pl.ds(0, _CHUNK)],
                                    acc.at[idx_b[b].at[0]], add=True)
                    pltpu.sync_copy(rows_b[b].at[pl.ds(_CHUNK, _CHUNK)],
                                    acc.at[idx_b[b].at[1]], add=True)
                issue_t = t + dt + 2
                issue(issue_t, b)

        if rem_full:
            @pl.when(wid == _NW - 2)
            def _():
                base = off + npairs * pair_rows
                pltpu.sync_copy(idx_hbm.at[pl.ds(base, _CHUNK)], idx0.at[0])
                pltpu.sync_copy(emb_hbm.at[pl.ds(base, _CHUNK)],
                                rows0.at[pl.ds(0, _CHUNK)])
                pltpu.sync_copy(rows0.at[pl.ds(0, _CHUNK)],
                                acc.at[idx0.at[0]], add=True)

        if tail:
            @pl.when(wid == _NW - 1)
            def _():
                base = off + npairs * pair_rows + rem_full * _CHUNK
                pltpu.sync_copy(idx_hbm.at[pl.ds(base, tail)], idx_t)
                pltpu.sync_copy(emb_hbm.at[pl.ds(base, tail)],
                                rows1.at[pl.ds(0, tail)])
                pltpu.sync_copy(rows1.at[pl.ds(0, tail)],
                                acc.at[idx_t], add=True)

        plsc.subcore_barrier()
        row0 = s * rows_per_tile
        pltpu.sync_copy(acc.at[pl.ds(row0, rows_per_tile)],
                        out_hbm.at[c, pl.ds(row0, rows_per_tile)])

    k = pl.kernel(
        body,
        out_type=jax.ShapeDtypeStruct((_NC, B, D), jnp.float32),
        mesh=mesh,
        scratch_types=[
            pltpu.VMEM((pair_rows, D), jnp.float32),
            pltpu.VMEM((pair_rows, D), jnp.float32),
            pltpu.VMEM((2, _CHUNK), jnp.int32),
            pltpu.VMEM((2, _CHUNK), jnp.int32),
            pltpu.VMEM((max(tail, 8),), jnp.int32),
            pltpu.VMEM((rows_per_tile, D), jnp.float32),
            pltpu.VMEM_SHARED((B, D), jnp.float32),
            pltpu.SemaphoreType.DMA,
            pltpu.SemaphoreType.DMA,
        ],
    )
    return k(embeddings, idx_i32)


_TC_BLK = 2048


def _tc_partial_segsum(embeddings, idx2d, B, nrows):
    N, D = embeddings.shape
    nblk = nrows // _TC_BLK

    def body(idx_ref, emb_ref, o_ref, acc_ref):
        i = pl.program_id(0)

        @pl.when(i == 0)
        def _():
            acc_ref[...] = jnp.zeros_like(acc_ref)

        seg = lax.broadcasted_iota(jnp.int32, (B, _TC_BLK), 0)
        onehot = (seg == idx_ref[...]).astype(jnp.float32)
        acc_ref[...] += lax.dot_general(
            onehot, emb_ref[...], (((1,), (0,)), ((), ())),
            precision=lax.Precision.DEFAULT,
            preferred_element_type=jnp.float32)

        @pl.when(i == nblk - 1)
        def _():
            o_ref[...] = acc_ref[...]

    return pl.pallas_call(
        body,
        grid=(nblk,),
        in_specs=[
            pl.BlockSpec((1, _TC_BLK), lambda i: (0, i)),
            pl.BlockSpec((_TC_BLK, D), lambda i: (i, 0)),
        ],
        out_specs=pl.BlockSpec((B, D), lambda i: (0, 0)),
        out_shape=jax.ShapeDtypeStruct((B, D), jnp.float32),
        scratch_shapes=[pltpu.VMEM((B, D), jnp.float32)],
    )(idx2d, embeddings)


def _tc_mlp(partials, tc_partial, virtual_node, W1, b1, g1, be1,
            W2, b2, g2, be2):
    B, D = virtual_node.shape

    def body(p_ref, pt_ref, vn_ref, w1_ref, b1_ref, g1_ref, be1_ref,
             w2_ref, b2_ref, g2_ref, be2_ref, o_ref):
        vn = vn_ref[...] + (p_ref[0] + p_ref[1] + pt_ref[...])
        h = lax.dot_general(vn, w1_ref[...], (((1,), (0,)), ((), ())),
                            precision=lax.Precision.DEFAULT,
                            preferred_element_type=jnp.float32)
        h = h + b1_ref[...]
        mu = jnp.mean(h, axis=0, keepdims=True)
        var = jnp.mean((h - mu) * (h - mu), axis=0, keepdims=True)
        h = (h - mu) / jnp.sqrt(var + BN_EPS) * g1_ref[...] + be1_ref[...]
        h = jnp.maximum(h, 0.0)
        h = lax.dot_general(h, w2_ref[...], (((1,), (0,)), ((), ())),
                            precision=lax.Precision.DEFAULT,
                            preferred_element_type=jnp.float32)
        h = h + b2_ref[...]
        mu2 = jnp.mean(h, axis=0, keepdims=True)
        var2 = jnp.mean((h - mu2) * (h - mu2), axis=0, keepdims=True)
        h = (h - mu2) / jnp.sqrt(var2 + BN_EPS) * g2_ref[...] + be2_ref[...]
        o_ref[...] = jnp.maximum(h, 0.0)

    return pl.pallas_call(
        body,
        out_shape=jax.ShapeDtypeStruct((B, D), jnp.float32),
    )(partials, tc_partial, virtual_node, W1, b1.reshape(1, -1),
      g1.reshape(1, -1), be1.reshape(1, -1), W2, b2.reshape(1, -1),
      g2.reshape(1, -1), be2.reshape(1, -1))


_TC_NBLK = 20


def kernel(virtual_node, embeddings, batch_idx, W1, b1, g1, be1,
           W2, b2, g2, be2):
    B = virtual_node.shape[0]
    N = embeddings.shape[0]
    idx_i32 = batch_idx.astype(jnp.int32)
    tc_rows = min(_TC_NBLK * _TC_BLK, N // _TC_BLK * _TC_BLK)
    sc_partials = _sc_segment_sum(embeddings, idx_i32, B, tc_rows)
    tc_partial = _tc_partial_segsum(embeddings, idx_i32.reshape(1, -1), B,
                                    tc_rows)
    return _tc_mlp(sc_partials, tc_partial, virtual_node, W1, b1, g1, be1,
                   W2, b2, g2, be2)


# speedup vs baseline: 1.5937x; 1.0053x over previous
import jax
import jax.numpy as jnp
from jax import lax
from jax.experimental import pallas as pl
from jax.experimental.pallas import tpu as pltpu
from jax.experimental.pallas import tpu_sc as plsc

BN_EPS = 1e-5

_NC = 2
_NS = 16
_NW = _NC * _NS
_L = 16

_CHUNK = 128


def _sc_segment_sum(embeddings, idx_i32, B, off):
    N, D = embeddings.shape
    N = N - off
    pair_rows = 2 * _CHUNK
    npairs = N // pair_rows
    rem = N - npairs * pair_rows
    rem_full = rem // _CHUNK
    tail = rem - rem_full * _CHUNK
    steps = -(-npairs // _NW)
    rows_per_tile = B // _NS

    mesh = plsc.VectorSubcoreMesh(core_axis_name="c", subcore_axis_name="s")

    def body(emb_hbm, idx_hbm, out_hbm, rows0, rows1, idx0, idx1,
             idx_t, zbuf, acc, sem0, sem1):
        rows_b = (rows0, rows1)
        idx_b = (idx0, idx1)
        sems = (sem0, sem1)
        c = lax.axis_index("c")
        s = lax.axis_index("s")
        wid = c * _NS + s

        @pl.loop(0, rows_per_tile)
        def _(r):
            for j in range(D // _L):
                zbuf[r, pl.ds(j * _L, _L)] = jnp.zeros((_L,), jnp.float32)

        pltpu.sync_copy(zbuf, acc.at[pl.ds(s * rows_per_tile, rows_per_tile)])
        plsc.subcore_barrier()

        def gather_descs(pid, b):
            base = pl.multiple_of(off + pid * pair_rows, 8)
            return (
                (emb_hbm.at[pl.ds(base, pair_rows)], rows_b[b]),
                (idx_hbm.at[pl.ds(base, _CHUNK)], idx_b[b].at[0]),
                (idx_hbm.at[pl.ds(base + _CHUNK, _CHUNK)], idx_b[b].at[1]),
            )

        def issue(t, b):
            pid = wid + _NW * t

            @pl.when(pid < npairs)
            def _():
                for src, dst in gather_descs(pid, b):
                    pltpu.async_copy(src, dst, sems[b])

        issue(0, 0)
        issue(1, 1)

        @pl.loop(0, 2 * (-(-steps // 2)), step=2)
        def _(t):
            for dt in range(2):
                b = dt
                pid = wid + _NW * (t + dt)

                @pl.when(pid < npairs)
                def _():
                    for src, dst in gather_descs(pid, b):
                        pltpu.make_async_copy(src, dst, sems[b]).wait()
                    pltpu.sync_copy(rows_b[b].at[pl.ds(0, _CHUNK)],
                                    acc.at[idx_b[b].at[0]], add=True)
                    pltpu.sync_copy(rows_b[b].at[pl.ds(_CHUNK, _CHUNK)],
                                    acc.at[idx_b[b].at[1]], add=True)
                issue_t = t + dt + 2
                issue(issue_t, b)

        if rem_full:
            @pl.when(wid == _NW - 2)
            def _():
                base = off + npairs * pair_rows
                pltpu.sync_copy(idx_hbm.at[pl.ds(base, _CHUNK)], idx0.at[0])
                pltpu.sync_copy(emb_hbm.at[pl.ds(base, _CHUNK)],
                                rows0.at[pl.ds(0, _CHUNK)])
                pltpu.sync_copy(rows0.at[pl.ds(0, _CHUNK)],
                                acc.at[idx0.at[0]], add=True)

        if tail:
            @pl.when(wid == _NW - 1)
            def _():
                base = off + npairs * pair_rows + rem_full * _CHUNK
                pltpu.sync_copy(idx_hbm.at[pl.ds(base, tail)], idx_t)
                pltpu.sync_copy(emb_hbm.at[pl.ds(base, tail)],
                                rows1.at[pl.ds(0, tail)])
                pltpu.sync_copy(rows1.at[pl.ds(0, tail)],
                                acc.at[idx_t], add=True)

        plsc.subcore_barrier()
        row0 = s * rows_per_tile
        pltpu.sync_copy(acc.at[pl.ds(row0, rows_per_tile)],
                        out_hbm.at[c, pl.ds(row0, rows_per_tile)])

    k = pl.kernel(
        body,
        out_type=jax.ShapeDtypeStruct((_NC, B, D), jnp.float32),
        mesh=mesh,
        scratch_types=[
            pltpu.VMEM((pair_rows, D), jnp.float32),
            pltpu.VMEM((pair_rows, D), jnp.float32),
            pltpu.VMEM((2, _CHUNK), jnp.int32),
            pltpu.VMEM((2, _CHUNK), jnp.int32),
            pltpu.VMEM((max(tail, 8),), jnp.int32),
            pltpu.VMEM((rows_per_tile, D), jnp.float32),
            pltpu.VMEM_SHARED((B, D), jnp.float32),
            pltpu.SemaphoreType.DMA,
            pltpu.SemaphoreType.DMA,
        ],
    )
    return k(embeddings, idx_i32)


_TC_BLK = 2048


def _tc_partial_segsum(embeddings, idx2d, B, nrows):
    N, D = embeddings.shape
    nblk = nrows // _TC_BLK

    W = 64

    def body(idx_ref, emb_ref, o_ref, acc_ref):
        i = pl.program_id(0)

        @pl.when(i == 0)
        def _():
            acc_ref[...] = jnp.zeros_like(acc_ref)

        lo = idx_ref[0, 0]
        hi = idx_ref[0, _TC_BLK - 1]
        lo8 = jnp.minimum((lo // 8) * 8, B - W)

        @pl.when(hi - lo8 < W)
        def _():
            seg = lax.broadcasted_iota(jnp.int32, (W, _TC_BLK), 0) + lo8
            onehot = (seg == idx_ref[...]).astype(jnp.float32)
            acc_ref[pl.ds(lo8, W), :] += lax.dot_general(
                onehot, emb_ref[...], (((1,), (0,)), ((), ())),
                precision=lax.Precision.DEFAULT,
                preferred_element_type=jnp.float32)

        @pl.when(hi - lo8 >= W)
        def _():
            seg = lax.broadcasted_iota(jnp.int32, (B, _TC_BLK), 0)
            onehot = (seg == idx_ref[...]).astype(jnp.float32)
            acc_ref[...] += lax.dot_general(
                onehot, emb_ref[...], (((1,), (0,)), ((), ())),
                precision=lax.Precision.DEFAULT,
                preferred_element_type=jnp.float32)

        @pl.when(i == nblk - 1)
        def _():
            o_ref[...] = acc_ref[...]

    return pl.pallas_call(
        body,
        grid=(nblk,),
        in_specs=[
            pl.BlockSpec((1, _TC_BLK), lambda i: (0, i)),
            pl.BlockSpec((_TC_BLK, D), lambda i: (i, 0)),
        ],
        out_specs=pl.BlockSpec((B, D), lambda i: (0, 0)),
        out_shape=jax.ShapeDtypeStruct((B, D), jnp.float32),
        scratch_shapes=[pltpu.VMEM((B, D), jnp.float32)],
    )(idx2d, embeddings)


def _tc_mlp(partials, tc_partial, virtual_node, W1, b1, g1, be1,
            W2, b2, g2, be2):
    B, D = virtual_node.shape

    def body(p_ref, pt_ref, vn_ref, w1_ref, b1_ref, g1_ref, be1_ref,
             w2_ref, b2_ref, g2_ref, be2_ref, o_ref):
        vn = vn_ref[...] + (p_ref[0] + p_ref[1] + pt_ref[...])
        h = lax.dot_general(vn, w1_ref[...], (((1,), (0,)), ((), ())),
                            precision=lax.Precision.DEFAULT,
                            preferred_element_type=jnp.float32)
        h = h + b1_ref[...]
        mu = jnp.mean(h, axis=0, keepdims=True)
        var = jnp.mean((h - mu) * (h - mu), axis=0, keepdims=True)
        h = (h - mu) / jnp.sqrt(var + BN_EPS) * g1_ref[...] + be1_ref[...]
        h = jnp.maximum(h, 0.0)
        h = lax.dot_general(h, w2_ref[...], (((1,), (0,)), ((), ())),
                            precision=lax.Precision.DEFAULT,
                            preferred_element_type=jnp.float32)
        h = h + b2_ref[...]
        mu2 = jnp.mean(h, axis=0, keepdims=True)
        var2 = jnp.mean((h - mu2) * (h - mu2), axis=0, keepdims=True)
        h = (h - mu2) / jnp.sqrt(var2 + BN_EPS) * g2_ref[...] + be2_ref[...]
        o_ref[...] = jnp.maximum(h, 0.0)

    return pl.pallas_call(
        body,
        out_shape=jax.ShapeDtypeStruct((B, D), jnp.float32),
    )(partials, tc_partial, virtual_node, W1, b1.reshape(1, -1),
      g1.reshape(1, -1), be1.reshape(1, -1), W2, b2.reshape(1, -1),
      g2.reshape(1, -1), be2.reshape(1, -1))


_TC_NBLK = 20


def kernel(virtual_node, embeddings, batch_idx, W1, b1, g1, be1,
           W2, b2, g2, be2):
    B = virtual_node.shape[0]
    N = embeddings.shape[0]
    idx_i32 = batch_idx.astype(jnp.int32)
    tc_rows = min(_TC_NBLK * _TC_BLK, N // _TC_BLK * _TC_BLK)
    sc_partials = _sc_segment_sum(embeddings, idx_i32, B, tc_rows)
    tc_partial = _tc_partial_segsum(embeddings, idx_i32.reshape(1, -1), B,
                                    tc_rows)
    return _tc_mlp(sc_partials, tc_partial, virtual_node, W1, b1, g1, be1,
                   W2, b2, g2, be2)


# speedup vs baseline: 1.5984x; 1.0029x over previous
import jax
import jax.numpy as jnp
from jax import lax
from jax.experimental import pallas as pl
from jax.experimental.pallas import tpu as pltpu
from jax.experimental.pallas import tpu_sc as plsc

BN_EPS = 1e-5

_NC = 2
_NS = 16
_NW = _NC * _NS
_L = 16

_CHUNK = 128


def _sc_segment_sum(embeddings, idx_i32, B, off):
    N, D = embeddings.shape
    N = N - off
    pair_rows = 2 * _CHUNK
    npairs = N // pair_rows
    rem = N - npairs * pair_rows
    rem_full = rem // _CHUNK
    tail = rem - rem_full * _CHUNK
    steps = -(-npairs // _NW)
    rows_per_tile = B // _NS

    mesh = plsc.VectorSubcoreMesh(core_axis_name="c", subcore_axis_name="s")

    def body(emb_hbm, idx_hbm, out_hbm, rows0, rows1, idx0, idx1,
             idx_t, zbuf, acc, sem0, sem1):
        rows_b = (rows0, rows1)
        idx_b = (idx0, idx1)
        sems = (sem0, sem1)
        c = lax.axis_index("c")
        s = lax.axis_index("s")
        wid = c * _NS + s

        @pl.loop(0, rows_per_tile)
        def _(r):
            for j in range(D // _L):
                zbuf[r, pl.ds(j * _L, _L)] = jnp.zeros((_L,), jnp.float32)

        pltpu.sync_copy(zbuf, acc.at[pl.ds(s * rows_per_tile, rows_per_tile)])
        plsc.subcore_barrier()

        def gather_descs(pid, b):
            base = pl.multiple_of(off + pid * pair_rows, 8)
            return (
                (emb_hbm.at[pl.ds(base, pair_rows)], rows_b[b]),
                (idx_hbm.at[pl.ds(base, _CHUNK)], idx_b[b].at[0]),
                (idx_hbm.at[pl.ds(base + _CHUNK, _CHUNK)], idx_b[b].at[1]),
            )

        def issue(t, b):
            pid = wid + _NW * t

            @pl.when(pid < npairs)
            def _():
                for src, dst in gather_descs(pid, b):
                    pltpu.async_copy(src, dst, sems[b])

        issue(0, 0)
        issue(1, 1)

        @pl.loop(0, 2 * (-(-steps // 2)), step=2)
        def _(t):
            for dt in range(2):
                b = dt
                pid = wid + _NW * (t + dt)

                @pl.when(pid < npairs)
                def _():
                    for src, dst in gather_descs(pid, b):
                        pltpu.make_async_copy(src, dst, sems[b]).wait()
                    pltpu.sync_copy(rows_b[b].at[pl.ds(0, _CHUNK)],
                                    acc.at[idx_b[b].at[0]], add=True)
                    pltpu.sync_copy(rows_b[b].at[pl.ds(_CHUNK, _CHUNK)],
                                    acc.at[idx_b[b].at[1]], add=True)
                issue_t = t + dt + 2
                issue(issue_t, b)

        if rem_full:
            @pl.when(wid == _NW - 2)
            def _():
                base = off + npairs * pair_rows
                pltpu.sync_copy(idx_hbm.at[pl.ds(base, _CHUNK)], idx0.at[0])
                pltpu.sync_copy(emb_hbm.at[pl.ds(base, _CHUNK)],
                                rows0.at[pl.ds(0, _CHUNK)])
                pltpu.sync_copy(rows0.at[pl.ds(0, _CHUNK)],
                                acc.at[idx0.at[0]], add=True)

        if tail:
            @pl.when(wid == _NW - 1)
            def _():
                base = off + npairs * pair_rows + rem_full * _CHUNK
                pltpu.sync_copy(idx_hbm.at[pl.ds(base, tail)], idx_t)
                pltpu.sync_copy(emb_hbm.at[pl.ds(base, tail)],
                                rows1.at[pl.ds(0, tail)])
                pltpu.sync_copy(rows1.at[pl.ds(0, tail)],
                                acc.at[idx_t], add=True)

        plsc.subcore_barrier()
        row0 = s * rows_per_tile
        pltpu.sync_copy(acc.at[pl.ds(row0, rows_per_tile)],
                        out_hbm.at[c, pl.ds(row0, rows_per_tile)])

    k = pl.kernel(
        body,
        out_type=jax.ShapeDtypeStruct((_NC, B, D), jnp.float32),
        mesh=mesh,
        scratch_types=[
            pltpu.VMEM((pair_rows, D), jnp.float32),
            pltpu.VMEM((pair_rows, D), jnp.float32),
            pltpu.VMEM((2, _CHUNK), jnp.int32),
            pltpu.VMEM((2, _CHUNK), jnp.int32),
            pltpu.VMEM((max(tail, 8),), jnp.int32),
            pltpu.VMEM((rows_per_tile, D), jnp.float32),
            pltpu.VMEM_SHARED((B, D), jnp.float32),
            pltpu.SemaphoreType.DMA,
            pltpu.SemaphoreType.DMA,
        ],
    )
    return k(embeddings, idx_i32)


_TC_BLK = 2048


def _tc_partial_segsum(embeddings, idx2d, B, nrows):
    N, D = embeddings.shape
    nblk = nrows // _TC_BLK

    W = 64
    H = _TC_BLK // 2

    def body(idx_ref, emb0_ref, emb1_ref, o_ref, acc_ref):
        i = pl.program_id(0)

        @pl.when(i == 0)
        def _():
            acc_ref[...] = jnp.zeros_like(acc_ref)

        def seg_dot(onehot):
            return (lax.dot_general(
                        onehot[:, :H], emb0_ref[...], (((1,), (0,)), ((), ())),
                        precision=lax.Precision.DEFAULT,
                        preferred_element_type=jnp.float32)
                    + lax.dot_general(
                        onehot[:, H:], emb1_ref[...], (((1,), (0,)), ((), ())),
                        precision=lax.Precision.DEFAULT,
                        preferred_element_type=jnp.float32))

        lo = idx_ref[0, 0]
        hi = idx_ref[0, _TC_BLK - 1]
        lo8 = jnp.minimum((lo // 8) * 8, B - W)

        @pl.when(hi - lo8 < W)
        def _():
            seg = lax.broadcasted_iota(jnp.int32, (W, _TC_BLK), 0) + lo8
            onehot = (seg == idx_ref[...]).astype(jnp.float32)
            acc_ref[pl.ds(lo8, W), :] += seg_dot(onehot)

        @pl.when(hi - lo8 >= W)
        def _():
            seg = lax.broadcasted_iota(jnp.int32, (B, _TC_BLK), 0)
            onehot = (seg == idx_ref[...]).astype(jnp.float32)
            acc_ref[...] += seg_dot(onehot)

        @pl.when(i == nblk - 1)
        def _():
            o_ref[...] = acc_ref[...]

    return pl.pallas_call(
        body,
        grid=(nblk,),
        in_specs=[
            pl.BlockSpec((1, _TC_BLK), lambda i: (0, i)),
            pl.BlockSpec((_TC_BLK // 2, D), lambda i: (2 * i, 0)),
            pl.BlockSpec((_TC_BLK // 2, D), lambda i: (2 * i + 1, 0)),
        ],
        out_specs=pl.BlockSpec((B, D), lambda i: (0, 0)),
        out_shape=jax.ShapeDtypeStruct((B, D), jnp.float32),
        scratch_shapes=[pltpu.VMEM((B, D), jnp.float32)],
    )(idx2d, embeddings, embeddings)


def _tc_mlp(partials, tc_partial, virtual_node, W1, b1, g1, be1,
            W2, b2, g2, be2):
    B, D = virtual_node.shape

    def body(p_ref, pt_ref, vn_ref, w1_ref, b1_ref, g1_ref, be1_ref,
             w2_ref, b2_ref, g2_ref, be2_ref, o_ref):
        vn = vn_ref[...] + (p_ref[0] + p_ref[1] + pt_ref[...])
        h = lax.dot_general(vn, w1_ref[...], (((1,), (0,)), ((), ())),
                            precision=lax.Precision.DEFAULT,
                            preferred_element_type=jnp.float32)
        h = h + b1_ref[...]
        mu = jnp.mean(h, axis=0, keepdims=True)
        var = jnp.mean((h - mu) * (h - mu), axis=0, keepdims=True)
        h = (h - mu) / jnp.sqrt(var + BN_EPS) * g1_ref[...] + be1_ref[...]
        h = jnp.maximum(h, 0.0)
        h = lax.dot_general(h, w2_ref[...], (((1,), (0,)), ((), ())),
                            precision=lax.Precision.DEFAULT,
                            preferred_element_type=jnp.float32)
        h = h + b2_ref[...]
        mu2 = jnp.mean(h, axis=0, keepdims=True)
        var2 = jnp.mean((h - mu2) * (h - mu2), axis=0, keepdims=True)
        h = (h - mu2) / jnp.sqrt(var2 + BN_EPS) * g2_ref[...] + be2_ref[...]
        o_ref[...] = jnp.maximum(h, 0.0)

    return pl.pallas_call(
        body,
        out_shape=jax.ShapeDtypeStruct((B, D), jnp.float32),
    )(partials, tc_partial, virtual_node, W1, b1.reshape(1, -1),
      g1.reshape(1, -1), be1.reshape(1, -1), W2, b2.reshape(1, -1),
      g2.reshape(1, -1), be2.reshape(1, -1))


_TC_NBLK = 20


def kernel(virtual_node, embeddings, batch_idx, W1, b1, g1, be1,
           W2, b2, g2, be2):
    B = virtual_node.shape[0]
    N = embeddings.shape[0]
    idx_i32 = batch_idx.astype(jnp.int32)
    tc_rows = min(_TC_NBLK * _TC_BLK, N // _TC_BLK * _TC_BLK)
    sc_partials = _sc_segment_sum(embeddings, idx_i32, B, tc_rows)
    tc_partial = _tc_partial_segsum(embeddings, idx_i32.reshape(1, -1), B,
                                    tc_rows)
    return _tc_mlp(sc_partials, tc_partial, virtual_node, W1, b1, g1, be1,
                   W2, b2, g2, be2)
